# baseline (speedup 1.0000x reference)
import functools

import jax
import jax.numpy as jnp
from jax import lax
from jax.experimental import pallas as pl
from jax.experimental.pallas import tpu as pltpu
from jax.experimental.pallas import tpu_sc as plsc

N = 10000
D = 128
H = 128
A = 256
E = 320000

NC = 2
NS = 16
EW = E // NS
NST = 2
K = 125
NCHUNK = EW // (NST * K)
NB = 4
NPAD = 10240
RPT = NPAD // NS
HH = H // 2

_mesh = plsc.VectorSubcoreMesh(core_axis_name="c", subcore_axis_name="s")


@functools.partial(
    pl.kernel,
    out_type=jax.ShapeDtypeStruct((NC, NPAD), jnp.float32),
    mesh=_mesh,
    scratch_types=[
        pltpu.VMEM((NCHUNK, K), jnp.int32),
        pltpu.VMEM((K,), jnp.float32),
        pltpu.VMEM((RPT,), jnp.float32),
        pltpu.VMEM_SHARED((NPAD,), jnp.float32),
        pltpu.SemaphoreType.DMA,
    ],
    compiler_params=pltpu.CompilerParams(use_tc_tiling_on_sc=False),
)
def _sc_degree(dst_hbm, ones_hbm, zeros_hbm, degp_hbm,
               dst_all, ones_v, zrow_v, acc_sh, sem):
    c = lax.axis_index("c")
    s = lax.axis_index("s")
    pltpu.sync_copy(zeros_hbm, zrow_v)
    pltpu.sync_copy(zrow_v, acc_sh.at[pl.ds(s * RPT, RPT)])
    pltpu.sync_copy(ones_hbm, ones_v)
    pltpu.sync_copy(dst_hbm.at[s, c], dst_all)
    plsc.subcore_barrier()

    def group(gi, carry):
        for b in range(5):
            pltpu.async_copy(ones_v, acc_sh.at[dst_all.at[gi * 5 + b]], sem,
                             add=True)
        for b in range(5):
            pltpu.make_async_copy(ones_v, acc_sh.at[dst_all.at[gi * 5 + b]],
                                  sem).wait()
        return carry

    lax.fori_loop(0, NCHUNK // 5, group, 0)
    plsc.subcore_barrier()
    pltpu.sync_copy(acc_sh.at[pl.ds(s * RPT, RPT)],
                    degp_hbm.at[c, pl.ds(s * RPT, RPT)])


@functools.partial(
    pl.kernel,
    out_type=jax.ShapeDtypeStruct((NC, NPAD, HH), jnp.float32),
    mesh=_mesh,
    scratch_types=[
        pltpu.VMEM((NCHUNK, K), jnp.int32),
        pltpu.VMEM((NCHUNK, K), jnp.int32),
        [pltpu.VMEM((K, HH), jnp.float32)] * NB,
        pltpu.VMEM_SHARED((NPAD, HH), jnp.float32),
        [pltpu.SemaphoreType.DMA] * NB,
    ],
    compiler_params=pltpu.CompilerParams(use_tc_tiling_on_sc=False),
)
def _sc_scatter(g_hbm, src_hbm, dst_hbm, zeros_hbm, part_hbm,
                src_all, dst_all, rows, acc_sh, sg):
    c = lax.axis_index("c")
    s = lax.axis_index("s")
    for z in range(RPT // 128):
        pltpu.sync_copy(zeros_hbm,
                        acc_sh.at[pl.ds(s * RPT + z * 128, 128)])
    plsc.subcore_barrier()

    for st in range(NST):
        pltpu.sync_copy(src_hbm.at[s, st], src_all)
        pltpu.sync_copy(dst_hbm.at[s, st], dst_all)

        def gather(j, b):
            pltpu.async_copy(g_hbm.at[c].at[src_all.at[j]], rows[b], sg[b])

        def gather_wait(j, b):
            pltpu.make_async_copy(g_hbm.at[c].at[src_all.at[j]], rows[b],
                                  sg[b]).wait()

        for b in range(NB):
            gather(b, b)

        def stepgroup(i, carry):
            j0 = i * NB
            for u in range(NB):
                j = j0 + u
                gather_wait(j, u)
                pltpu.sync_copy(rows[u], acc_sh.at[dst_all.at[j]], add=True)
                gather(j + NB, u)
            return carry

        lax.fori_loop(0, NCHUNK // NB - 1, stepgroup, 0)
        for u in range(NB):
            j = NCHUNK - NB + u
            gather_wait(j, u)
            pltpu.sync_copy(rows[u], acc_sh.at[dst_all.at[j]], add=True)

    plsc.subcore_barrier()
    pltpu.sync_copy(acc_sh.at[pl.ds(s * RPT, RPT)],
                    part_hbm.at[c, pl.ds(s * RPT, RPT)])


def _tc_mm_body(x_ref, w1_ref, h1_ref):
    h1_ref[...] = jnp.dot(x_ref[...], w1_ref[...],
                          preferred_element_type=jnp.float32)


def _tc1_body(deg0_ref, deg1_ref, h1_ref, dinv_ref, g_ref):
    deg = deg0_ref[...] + deg1_ref[...] + 1.0
    dinv = lax.rsqrt(deg)
    dinv_ref[...] = dinv
    g = h1_ref[...] * dinv
    g_ref[0] = g[:, :HH]
    g_ref[1] = g[:, HH:]


def _combine(p_ref, h, res, dinv, b):
    ssum = jnp.concatenate([p_ref[0, :N, :], p_ref[1, :N, :]], axis=1)
    agg = ssum * dinv + h * (dinv * dinv)
    return jnp.maximum(agg + b + res, 0.0)


def _tc2a_body(p_ref, h_ref, res_ref, dinv_ref, b_ref, a_ref):
    a_ref[...] = _combine(p_ref, h_ref[...], res_ref[...], dinv_ref[...],
                          b_ref[...])


def _tc2b_body(a_ref, w2_ref, dinv_ref, h2_ref, g_ref):
    dinv = dinv_ref[...]
    h2 = jnp.dot(a_ref[...], w2_ref[...], preferred_element_type=jnp.float32)
    h2_ref[...] = h2
    g2 = h2 * dinv
    g_ref[0] = g2[:, :HH]
    g_ref[1] = g2[:, HH:]


def _tc3_body(p_ref, h_ref, res_ref, dinv_ref, b_ref,
              wa_ref, ba_ref, wc_ref, bc_ref, logits_ref, value_ref):
    a = _combine(p_ref, h_ref[...], res_ref[...], dinv_ref[...], b_ref[...])
    pooled = jnp.mean(a, axis=0, keepdims=True)
    logits_ref[...] = (
        jnp.dot(pooled, wa_ref[...], preferred_element_type=jnp.float32)
        + ba_ref[...])
    value_ref[...] = (
        jnp.dot(pooled, wc_ref[...], preferred_element_type=jnp.float32)
        + bc_ref[...])


def kernel(x, edge_index, W1, b1, W2, b2, Wa, ba, Wc, bc):
    eidx = edge_index.reshape(2, NS, NST, NCHUNK, K)
    src = eidx[0]
    dst = eidx[1]
    zeros_row = jnp.zeros((RPT,), jnp.float32)
    ones_k = jnp.ones((K,), jnp.float32)
    zeros_rows = jnp.zeros((128, HH), jnp.float32)

    degp = _sc_degree(dst, ones_k, zeros_row)
    deg0 = degp[0, :N].reshape(N, 1)
    deg1 = degp[1, :N].reshape(N, 1)

    h1 = pl.pallas_call(
        _tc_mm_body,
        out_shape=jax.ShapeDtypeStruct((N, H), jnp.float32),
    )(x, W1)

    dinv, g1 = pl.pallas_call(
        _tc1_body,
        out_shape=[jax.ShapeDtypeStruct((N, 1), jnp.float32),
                   jax.ShapeDtypeStruct((NC, N, HH), jnp.float32)],
    )(deg0, deg1, h1)

    p1 = _sc_scatter(g1, src, dst, zeros_rows)
    a1 = pl.pallas_call(
        _tc2a_body,
        out_shape=jax.ShapeDtypeStruct((N, H), jnp.float32),
    )(p1, h1, x, dinv, b1.reshape(1, H))
    h2, g2 = pl.pallas_call(
        _tc2b_body,
        out_shape=[jax.ShapeDtypeStruct((N, H), jnp.float32),
                   jax.ShapeDtypeStruct((NC, N, HH), jnp.float32)],
    )(a1, W2, dinv)

    p2 = _sc_scatter(g2, src, dst, zeros_rows)
    logits, value = pl.pallas_call(
        _tc3_body,
        out_shape=[jax.ShapeDtypeStruct((1, A), jnp.float32),
                   jax.ShapeDtypeStruct((1, 1), jnp.float32)],
    )(p2, h2, a1, dinv, b2.reshape(1, H),
      Wa, ba.reshape(1, A), Wc, bc.reshape(1, 1))

    return (logits, value)


# speedup vs baseline: 1.0079x; 1.0079x over previous
import functools

import jax
import jax.numpy as jnp
from jax import lax
from jax.experimental import pallas as pl
from jax.experimental.pallas import tpu as pltpu
from jax.experimental.pallas import tpu_sc as plsc

N = 10000
D = 128
H = 128
A = 256
E = 320000

NC = 2
NS = 16
EW = E // NS
NST = 2
K = 100
NCHUNK = EW // (NST * K)
NB = 4
NPAD = 10240
RPT = NPAD // NS
HH = H // 2

_mesh = plsc.VectorSubcoreMesh(core_axis_name="c", subcore_axis_name="s")


@functools.partial(
    pl.kernel,
    out_type=jax.ShapeDtypeStruct((NC, NPAD), jnp.float32),
    mesh=_mesh,
    scratch_types=[
        pltpu.VMEM((NCHUNK, K), jnp.int32),
        pltpu.VMEM((K,), jnp.float32),
        pltpu.VMEM((RPT,), jnp.float32),
        pltpu.VMEM_SHARED((NPAD,), jnp.float32),
        pltpu.SemaphoreType.DMA,
    ],
    compiler_params=pltpu.CompilerParams(use_tc_tiling_on_sc=False),
)
def _sc_degree(dst_hbm, ones_hbm, zeros_hbm, degp_hbm,
               dst_all, ones_v, zrow_v, acc_sh, sem):
    c = lax.axis_index("c")
    s = lax.axis_index("s")
    pltpu.sync_copy(zeros_hbm, zrow_v)
    pltpu.sync_copy(zrow_v, acc_sh.at[pl.ds(s * RPT, RPT)])
    pltpu.sync_copy(ones_hbm, ones_v)
    pltpu.sync_copy(dst_hbm.at[s, c], dst_all)
    plsc.subcore_barrier()

    def group(gi, carry):
        for b in range(5):
            pltpu.async_copy(ones_v, acc_sh.at[dst_all.at[gi * 5 + b]], sem,
                             add=True)
        for b in range(5):
            pltpu.make_async_copy(ones_v, acc_sh.at[dst_all.at[gi * 5 + b]],
                                  sem).wait()
        return carry

    lax.fori_loop(0, NCHUNK // 5, group, 0)
    plsc.subcore_barrier()
    pltpu.sync_copy(acc_sh.at[pl.ds(s * RPT, RPT)],
                    degp_hbm.at[c, pl.ds(s * RPT, RPT)])


@functools.partial(
    pl.kernel,
    out_type=jax.ShapeDtypeStruct((NC, NPAD, HH), jnp.float32),
    mesh=_mesh,
    scratch_types=[
        pltpu.VMEM((NCHUNK, K), jnp.int32),
        pltpu.VMEM((NCHUNK, K), jnp.int32),
        [pltpu.VMEM((K, HH), jnp.float32)] * NB,
        pltpu.VMEM_SHARED((NPAD, HH), jnp.float32),
        [pltpu.SemaphoreType.DMA] * NB,
    ],
    compiler_params=pltpu.CompilerParams(use_tc_tiling_on_sc=False),
)
def _sc_scatter(g_hbm, src_hbm, dst_hbm, zeros_hbm, part_hbm,
                src_all, dst_all, rows, acc_sh, sg):
    c = lax.axis_index("c")
    s = lax.axis_index("s")
    for z in range(RPT // 128):
        pltpu.sync_copy(zeros_hbm,
                        acc_sh.at[pl.ds(s * RPT + z * 128, 128)])
    plsc.subcore_barrier()

    for st in range(NST):
        pltpu.sync_copy(src_hbm.at[s, st], src_all)
        pltpu.sync_copy(dst_hbm.at[s, st], dst_all)

        def gather(j, b):
            pltpu.async_copy(g_hbm.at[c].at[src_all.at[j]], rows[b], sg[b])

        def gather_wait(j, b):
            pltpu.make_async_copy(g_hbm.at[c].at[src_all.at[j]], rows[b],
                                  sg[b]).wait()

        for b in range(NB):
            gather(b, b)

        def stepgroup(i, carry):
            j0 = i * NB
            for u in range(NB):
                j = j0 + u
                gather_wait(j, u)
                pltpu.sync_copy(rows[u], acc_sh.at[dst_all.at[j]], add=True)
                gather(j + NB, u)
            return carry

        lax.fori_loop(0, NCHUNK // NB - 1, stepgroup, 0)
        for u in range(NB):
            j = NCHUNK - NB + u
            gather_wait(j, u)
            pltpu.sync_copy(rows[u], acc_sh.at[dst_all.at[j]], add=True)

    plsc.subcore_barrier()
    pltpu.sync_copy(acc_sh.at[pl.ds(s * RPT, RPT)],
                    part_hbm.at[c, pl.ds(s * RPT, RPT)])


def _tc_mm_body(x_ref, w1_ref, h1_ref):
    h1_ref[...] = jnp.dot(x_ref[...], w1_ref[...],
                          preferred_element_type=jnp.float32)


def _tc1_body(deg0_ref, deg1_ref, h1_ref, dinv_ref, g_ref):
    deg = deg0_ref[...] + deg1_ref[...] + 1.0
    dinv = lax.rsqrt(deg)
    dinv_ref[...] = dinv
    g = h1_ref[...] * dinv
    g_ref[0] = g[:, :HH]
    g_ref[1] = g[:, HH:]


def _combine(p_ref, h, res, dinv, b):
    ssum = jnp.concatenate([p_ref[0, :N, :], p_ref[1, :N, :]], axis=1)
    agg = ssum * dinv + h * (dinv * dinv)
    return jnp.maximum(agg + b + res, 0.0)


def _tc2a_body(p_ref, h_ref, res_ref, dinv_ref, b_ref, a_ref):
    a_ref[...] = _combine(p_ref, h_ref[...], res_ref[...], dinv_ref[...],
                          b_ref[...])


def _tc2b_body(a_ref, w2_ref, dinv_ref, h2_ref, g_ref):
    dinv = dinv_ref[...]
    h2 = jnp.dot(a_ref[...], w2_ref[...], preferred_element_type=jnp.float32)
    h2_ref[...] = h2
    g2 = h2 * dinv
    g_ref[0] = g2[:, :HH]
    g_ref[1] = g2[:, HH:]


def _tc3_body(p_ref, h_ref, res_ref, dinv_ref, b_ref,
              wa_ref, ba_ref, wc_ref, bc_ref, logits_ref, value_ref):
    a = _combine(p_ref, h_ref[...], res_ref[...], dinv_ref[...], b_ref[...])
    pooled = jnp.mean(a, axis=0, keepdims=True)
    logits_ref[...] = (
        jnp.dot(pooled, wa_ref[...], preferred_element_type=jnp.float32)
        + ba_ref[...])
    value_ref[...] = (
        jnp.dot(pooled, wc_ref[...], preferred_element_type=jnp.float32)
        + bc_ref[...])


def kernel(x, edge_index, W1, b1, W2, b2, Wa, ba, Wc, bc):
    eidx = edge_index.reshape(2, NS, NST, NCHUNK, K)
    src = eidx[0]
    dst = eidx[1]
    zeros_row = jnp.zeros((RPT,), jnp.float32)
    ones_k = jnp.ones((K,), jnp.float32)
    zeros_rows = jnp.zeros((128, HH), jnp.float32)

    degp = _sc_degree(dst, ones_k, zeros_row)
    deg0 = degp[0, :N].reshape(N, 1)
    deg1 = degp[1, :N].reshape(N, 1)

    h1 = pl.pallas_call(
        _tc_mm_body,
        out_shape=jax.ShapeDtypeStruct((N, H), jnp.float32),
    )(x, W1)

    dinv, g1 = pl.pallas_call(
        _tc1_body,
        out_shape=[jax.ShapeDtypeStruct((N, 1), jnp.float32),
                   jax.ShapeDtypeStruct((NC, N, HH), jnp.float32)],
    )(deg0, deg1, h1)

    p1 = _sc_scatter(g1, src, dst, zeros_rows)
    a1 = pl.pallas_call(
        _tc2a_body,
        out_shape=jax.ShapeDtypeStruct((N, H), jnp.float32),
    )(p1, h1, x, dinv, b1.reshape(1, H))
    h2, g2 = pl.pallas_call(
        _tc2b_body,
        out_shape=[jax.ShapeDtypeStruct((N, H), jnp.float32),
                   jax.ShapeDtypeStruct((NC, N, HH), jnp.float32)],
    )(a1, W2, dinv)

    p2 = _sc_scatter(g2, src, dst, zeros_rows)
    logits, value = pl.pallas_call(
        _tc3_body,
        out_shape=[jax.ShapeDtypeStruct((1, A), jnp.float32),
                   jax.ShapeDtypeStruct((1, 1), jnp.float32)],
    )(p2, h2, a1, dinv, b2.reshape(1, H),
      Wa, ba.reshape(1, A), Wc, bc.reshape(1, 1))

    return (logits, value)
